# Initial kernel scaffold; baseline (speedup 1.0000x reference)
import functools
import math

import jax
import jax.numpy as jnp
from jax import lax
from jax.experimental import pallas as pl
from jax.experimental.pallas import tpu as pltpu
from jax.experimental.pallas import tpu_sc as plsc

N = 8192
K = 16
D = 256
D_FF = 1024
NHEAD = 8
DH = 32
POS_HID = 32

KNN_R = 256
ATT_R = 128
XPAD = 16
TW = D + XPAD

_NEG = jnp.float32(-1e9)


def _knn_body(own_ref, allx_ref, inds_ref, d2_ref):
    d2 = None
    for c in range(3):
        oc = own_ref[:, c:c + 1]
        ac = allx_ref[c:c + 1, :]
        diff = oc - ac
        d2 = diff * diff if d2 is None else d2 + diff * diff
    iota = lax.broadcasted_iota(jnp.int32, (KNN_R, N), 1)
    big = jnp.int32(1 << 30)
    for t in range(K):
        m = jnp.min(d2, axis=1, keepdims=True)
        idx = jnp.min(jnp.where(d2 == m, iota, big), axis=1, keepdims=True)
        inds_ref[:, t:t + 1] = idx
        d2_ref[:, t:t + 1] = m
        d2 = jnp.where(iota == idx, jnp.float32(jnp.inf), d2)


def _run_knn(xyz_pad, xyz_t):
    grid = N // KNN_R
    return pl.pallas_call(
        _knn_body,
        grid=(grid,),
        in_specs=[
            pl.BlockSpec((KNN_R, 8), lambda i: (i, 0)),
            pl.BlockSpec((8, N), lambda i: (0, 0)),
        ],
        out_specs=[
            pl.BlockSpec((KNN_R, K), lambda i: (i, 0)),
            pl.BlockSpec((KNN_R, K), lambda i: (i, 0)),
        ],
        out_shape=[
            jax.ShapeDtypeStruct((N, K), jnp.int32),
            jax.ShapeDtypeStruct((N, K), jnp.float32),
        ],
    )(xyz_pad, xyz_t)


_SC_CHUNK = 128


def _sc_gather_body(table_hbm, idx_hbm, out_hbm, idxv, rows0, rows1, sem0, sem1):
    nc = 2
    ns = 16
    wid = lax.axis_index("s") * nc + lax.axis_index("c")
    b_per_w = (N * K) // (nc * ns)
    nchunk = b_per_w // _SC_CHUNK
    base = wid * b_per_w
    pltpu.sync_copy(idx_hbm.at[pl.ds(base, b_per_w)], idxv)

    rows = (rows0, rows1)
    sems = (sem0, sem1)

    def start(g, slot):
        pltpu.async_copy(
            table_hbm.at[idxv.at[pl.ds(g * _SC_CHUNK, _SC_CHUNK)]],
            rows[slot], sems[slot])

    def drain(g, slot):
        pltpu.make_async_copy(
            table_hbm.at[idxv.at[pl.ds(0, _SC_CHUNK)]], rows[slot], sems[slot]
        ).wait()
        pltpu.sync_copy(rows[slot],
                        out_hbm.at[pl.ds(base + g * _SC_CHUNK, _SC_CHUNK)])

    nchunk_s = nchunk
    start(0, 0)
    for g in range(nchunk_s):
        slot = g % 2
        if g + 1 < nchunk_s:
            start(g + 1, 1 - slot)
        drain(g, slot)


def _run_sc_gather(table, flat_inds):
    mesh = plsc.VectorSubcoreMesh(core_axis_name="c", subcore_axis_name="s")
    b_per_w = (N * K) // 32
    kern = functools.partial(
        pl.kernel,
        mesh=mesh,
        out_type=jax.ShapeDtypeStruct((N * K, TW), jnp.float32),
        scratch_types=[
            pltpu.VMEM((b_per_w,), jnp.int32),
            pltpu.VMEM((_SC_CHUNK, TW), jnp.float32),
            pltpu.VMEM((_SC_CHUNK, TW), jnp.float32),
            pltpu.SemaphoreType.DMA,
            pltpu.SemaphoreType.DMA,
        ],
    )(_sc_gather_body)
    return kern(table, flat_inds)


def _wprep_body(wkT_ref, pm_w2T_ref, pm_w2_ref, wq_ref, wv_ref, wo_ref,
                pm_b2_ref, bq_ref, bk_ref, bv_ref, bo_ref,
                wvo_ref, wpkT_ref, wpq_ref, cq_ref, ckb_ref, co_ref):
    f32 = jnp.float32
    wkT = wkT_ref[...]
    wpkT_ref[...] = jnp.dot(wkT, pm_w2T_ref[...], preferred_element_type=f32)
    wpq_ref[...] = jnp.dot(pm_w2_ref[...], wq_ref[...], preferred_element_type=f32)
    pm_b2 = pm_b2_ref[...]
    cq_ref[...] = jnp.dot(pm_b2, wq_ref[...], preferred_element_type=f32) + bq_ref[...]
    ckb_ref[...] = jnp.dot(pm_b2, wkT_ref[...].T, preferred_element_type=f32) + bk_ref[...]
    co_ref[...] = jnp.dot(bv_ref[...], wo_ref[...], preferred_element_type=f32) + bo_ref[...]
    wv = wv_ref[...]
    wo = wo_ref[...]
    for h in range(NHEAD):
        hs = slice(h * DH, (h + 1) * DH)
        wvo_ref[h * D:(h + 1) * D, :] = jnp.dot(
            wv[:, hs], wo[hs, :], preferred_element_type=f32)


def _run_wprep(wkT, pm_w2T, pm_w2, wq, wv, wo, pm_b2, bq, bk, bv, bo):
    return pl.pallas_call(
        _wprep_body,
        out_shape=[
            jax.ShapeDtypeStruct((NHEAD * D, D), jnp.float32),
            jax.ShapeDtypeStruct((D, POS_HID), jnp.float32),
            jax.ShapeDtypeStruct((POS_HID, D), jnp.float32),
            jax.ShapeDtypeStruct((1, D), jnp.float32),
            jax.ShapeDtypeStruct((1, D), jnp.float32),
            jax.ShapeDtypeStruct((1, D), jnp.float32),
        ],
    )(wkT, pm_w2T, pm_w2, wq, wv, wo, pm_b2, bq, bk, bv, bo)


def _ln(x, g, b):
    m = jnp.mean(x, axis=-1, keepdims=True)
    xc = x - m
    v = jnp.mean(xc * xc, axis=-1, keepdims=True)
    return xc * lax.rsqrt(v + 1e-5) * g + b


def _attn_body(feats_ref, ownx_ref, gat_ref, d2_ref,
               wq_ref, wkT_ref, wpkT_ref, wpq_ref, cq_ref, ckb_ref,
               wvo_ref, co_ref, pm_w1_ref, pm_b1_ref, pm_g_ref, pm_bt_ref,
               l1w_ref, l1b_ref, l2w_ref, l2b_ref,
               n1g_ref, n1b_ref, n2g_ref, n2b_ref, out_ref):
    f32 = jnp.float32
    R = ATT_R
    feats = feats_ref[...]
    gat = gat_ref[...]
    gf = gat[:, :D].reshape(R, K, D)
    gx = gat[:, D:].reshape(R, K, XPAD)

    hpre = None
    for c in range(3):
        relc = gx[:, :, c:c + 1] - ownx_ref[:, c:c + 1].reshape(R, 1, 1)
        w1c = pm_w1_ref[c:c + 1, :].reshape(1, 1, POS_HID)
        term = (relc * (1.0 / 10.0)) * w1c
        hpre = term if hpre is None else hpre + term
    hpre = hpre + pm_b1_ref[...].reshape(1, 1, POS_HID)
    hm = jnp.mean(hpre, axis=-1, keepdims=True)
    hc = hpre - hm
    hv = jnp.mean(hc * hc, axis=-1, keepdims=True)
    hmlp = hc * lax.rsqrt(hv + 1e-5) * pm_g_ref[...].reshape(1, 1, POS_HID) \
        + pm_bt_ref[...].reshape(1, 1, POS_HID)
    hmlp = jnp.maximum(hmlp, 0.0)

    h0 = hmlp[:, 0, :]
    qh = (jnp.dot(feats, wq_ref[...], preferred_element_type=f32)
          + jnp.dot(h0, wpq_ref[...], preferred_element_type=f32)
          + cq_ref[...])

    mask = jnp.sqrt(d2_ref[...]) > 0.5
    scale = 1.0 / math.sqrt(float(DH))
    ckb = ckb_ref[...]

    ctx_parts = []
    for h in range(NHEAD):
        hs = slice(h * DH, (h + 1) * DH)
        qh_h = qh[:, hs]
        u_h = jnp.dot(qh_h, wkT_ref[hs, :], preferred_element_type=f32)
        w_h = jnp.dot(qh_h, wpkT_ref[hs, :], preferred_element_type=f32)
        sb_h = jnp.sum(qh_h * ckb[:, hs], axis=-1, keepdims=True)
        s_feat = jnp.sum(u_h[:, None, :] * gf, axis=-1)
        s_pos = jnp.sum(w_h[:, None, :] * hmlp, axis=-1)
        s = (s_feat + s_pos + sb_h) * scale
        s = jnp.where(mask, _NEG, s)
        smax = jnp.max(s, axis=-1, keepdims=True)
        e = jnp.exp(s - smax)
        attn = e / jnp.sum(e, axis=-1, keepdims=True)
        ctx_h = jnp.sum(attn[:, :, None] * gf, axis=1)
        ctx_parts.append(ctx_h)
    ctx = jnp.concatenate(ctx_parts, axis=-1)

    o = jnp.dot(ctx, wvo_ref[...], preferred_element_type=f32) + co_ref[...]
    src = _ln(feats + o, n1g_ref[...], n1b_ref[...])
    ffp = jnp.dot(src, l1w_ref[...], preferred_element_type=f32) + l1b_ref[...]
    ff = ffp * 0.5 * (1.0 + lax.erf(ffp * (1.0 / math.sqrt(2.0))))
    ff = jnp.dot(ff, l2w_ref[...], preferred_element_type=f32) + l2b_ref[...]
    out_ref[...] = _ln(src + ff, n2g_ref[...], n2b_ref[...])


def _run_attn(feats, xyz_pad, gathered, d2,
              wq, wkT, wpkT, wpq, cq, ckb, wvo, co,
              pm_w1, pm_b1, pm_g, pm_bt, l1w, l1b, l2w, l2b,
              n1g, n1b, n2g, n2b):
    grid = N // ATT_R

    def full(shape):
        return pl.BlockSpec(shape, lambda i: tuple(0 for _ in shape))

    return pl.pallas_call(
        _attn_body,
        grid=(grid,),
        in_specs=[
            pl.BlockSpec((ATT_R, D), lambda i: (i, 0)),
            pl.BlockSpec((ATT_R, 8), lambda i: (i, 0)),
            pl.BlockSpec((ATT_R * K, TW), lambda i: (i, 0)),
            pl.BlockSpec((ATT_R, K), lambda i: (i, 0)),
            full((D, D)),
            full((D, D)),
            full((D, POS_HID)),
            full((POS_HID, D)),
            full((1, D)),
            full((1, D)),
            full((NHEAD * D, D)),
            full((1, D)),
            full((3, POS_HID)),
            full((1, POS_HID)),
            full((1, POS_HID)),
            full((1, POS_HID)),
            full((D, D_FF)),
            full((1, D_FF)),
            full((D_FF, D)),
            full((1, D)),
            full((1, D)),
            full((1, D)),
            full((1, D)),
            full((1, D)),
        ],
        out_specs=pl.BlockSpec((ATT_R, D), lambda i: (i, 0)),
        out_shape=jax.ShapeDtypeStruct((N, D), jnp.float32),
    )(feats, xyz_pad, gathered, d2,
      wq, wkT, wpkT, wpq, cq, ckb, wvo, co,
      pm_w1, pm_b1, pm_g, pm_bt, l1w, l1b, l2w, l2b,
      n1g, n1b, n2g, n2b)


def kernel(pts_feats, pts_xyz, pts_inds, pm_w1, pm_b1, pm_g, pm_bt, pm_w2,
           pm_b2, wq, bq, wk, bk, wv, bv, wo, bo, l1w, l1b, l2w, l2b,
           n1g, n1b, n2g, n2b):
    del pts_inds

    def row(v):
        return v.reshape(1, -1)

    xyz_pad8 = jnp.pad(pts_xyz, ((0, 0), (0, 8 - 3)))
    xyz_t = jnp.pad(pts_xyz.T, ((0, 8 - 3), (0, 0)))

    inds, d2 = _run_knn(xyz_pad8, xyz_t)

    table = jnp.concatenate(
        [pts_feats, jnp.pad(pts_xyz, ((0, 0), (0, XPAD - 3)))], axis=1)
    gathered = _run_sc_gather(table, inds.reshape(-1))

    wvo, wpkT, wpq, cq, ckb, co = _run_wprep(
        wk.T, pm_w2.T, pm_w2, wq, wv, wo,
        row(pm_b2), row(bq), row(bk), row(bv), row(bo))

    return _run_attn(
        pts_feats, xyz_pad8, gathered, d2,
        pts_feats_wq := wq, wk.T, wpkT, wpq, cq, ckb, wvo, co,
        pm_w1, row(pm_b1), row(pm_g), row(pm_bt),
        l1w, row(l1b), l2w, row(l2b),
        row(n1g), row(n1b), row(n2g), row(n2b))


# speedup vs baseline: 3.2285x; 3.2285x over previous
import functools
import math

import jax
import jax.numpy as jnp
from jax import lax
from jax.experimental import pallas as pl
from jax.experimental.pallas import tpu as pltpu
from jax.experimental.pallas import tpu_sc as plsc

N = 8192
K = 16
D = 256
D_FF = 1024
NHEAD = 8
DH = 32
POS_HID = 32

KNN_R = 64
ATT_R = 128
XPAD = 128
TW = D + XPAD

_NEG = -1e9


def _knn_body(own_ref, allx_ref, inds_ref, d2_ref, d2_s):
    R = KNN_R
    d2 = None
    for c in range(3):
        oc = own_ref[:, c:c + 1]
        ac = allx_ref[c:c + 1, :]
        diff = oc - ac
        d2 = diff * diff if d2 is None else d2 + diff * diff
    d2_s[...] = d2
    iota = lax.broadcasted_iota(jnp.int32, (R, N), 1)
    lane16 = lax.broadcasted_iota(jnp.int32, (R, K), 1)
    big = jnp.int32(1 << 30)

    def sel(t, carry):
        vacc, iacc = carry
        d = d2_s[...]
        m = jnp.min(d, axis=1, keepdims=True)
        idx = jnp.min(jnp.where(d == m, iota, big), axis=1, keepdims=True)
        d2_s[...] = jnp.where(iota == idx, jnp.float32(jnp.inf), d)
        vacc = jnp.where(lane16 == t, m, vacc)
        iacc = jnp.where(lane16 == t, idx, iacc)
        return vacc, iacc

    vacc, iacc = lax.fori_loop(
        0, K, sel,
        (jnp.zeros((R, K), jnp.float32), jnp.zeros((R, K), jnp.int32)))
    inds_ref[...] = iacc
    d2_ref[...] = vacc


def _run_knn(xyz_pad, xyz_t):
    grid = N // KNN_R
    return pl.pallas_call(
        _knn_body,
        grid=(grid,),
        in_specs=[
            pl.BlockSpec((KNN_R, 8), lambda i: (i, 0)),
            pl.BlockSpec((8, N), lambda i: (0, 0)),
        ],
        out_specs=[
            pl.BlockSpec((KNN_R, K), lambda i: (i, 0)),
            pl.BlockSpec((KNN_R, K), lambda i: (i, 0)),
        ],
        out_shape=[
            jax.ShapeDtypeStruct((N, K), jnp.int32),
            jax.ShapeDtypeStruct((N, K), jnp.float32),
        ],
        scratch_shapes=[pltpu.VMEM((KNN_R, N), jnp.float32)],
    )(xyz_pad, xyz_t)


_SC_CHUNK = 128


def _sc_gather_body(table_hbm, idx_hbm, out_hbm, idxv, rows0, rows1, sem0, sem1):
    nc = 2
    ns = 16
    wid = lax.axis_index("s") * nc + lax.axis_index("c")
    b_per_w = (N * K) // (nc * ns)
    nchunk = b_per_w // _SC_CHUNK
    base = wid * b_per_w
    pltpu.sync_copy(idx_hbm.at[pl.ds(base, b_per_w)], idxv)

    rows = (rows0, rows1)
    sems = (sem0, sem1)

    def start(g, slot):
        pltpu.async_copy(
            table_hbm.at[idxv.at[pl.ds(g * _SC_CHUNK, _SC_CHUNK)]],
            rows[slot], sems[slot])

    def drain(g, slot):
        pltpu.make_async_copy(
            table_hbm.at[idxv.at[pl.ds(0, _SC_CHUNK)]], rows[slot], sems[slot]
        ).wait()
        pltpu.sync_copy(rows[slot],
                        out_hbm.at[pl.ds(base + g * _SC_CHUNK, _SC_CHUNK)])

    start(0, 0)
    start(1, 1)

    def pair(g2, carry):
        g = g2 * 2
        drain(g, 0)

        @pl.when(g + 2 < nchunk)
        def _():
            start(g + 2, 0)

        drain(g + 1, 1)

        @pl.when(g + 3 < nchunk)
        def _():
            start(g + 3, 1)

        return carry

    lax.fori_loop(0, nchunk // 2, pair, 0)


def _run_sc_gather(table, flat_inds):
    mesh = plsc.VectorSubcoreMesh(core_axis_name="c", subcore_axis_name="s")
    b_per_w = (N * K) // 32
    kern = functools.partial(
        pl.kernel,
        mesh=mesh,
        out_type=jax.ShapeDtypeStruct((N * K, TW), jnp.float32),
        scratch_types=[
            pltpu.VMEM((b_per_w,), jnp.int32),
            pltpu.VMEM((_SC_CHUNK, TW), jnp.float32),
            pltpu.VMEM((_SC_CHUNK, TW), jnp.float32),
            pltpu.SemaphoreType.DMA,
            pltpu.SemaphoreType.DMA,
        ],
    )(_sc_gather_body)
    return kern(table, flat_inds)


def _wprep_body(wkT_ref, pm_w2T_ref, pm_w2_ref, wq_ref, wv_ref, wo_ref,
                pm_b2_ref, bq_ref, bk_ref, bv_ref, bo_ref,
                wvo_ref, wpkT_ref, wpq_ref, cq_ref, ckb_ref, co_ref):
    f32 = jnp.float32
    wkT = wkT_ref[...]
    wpkT_ref[...] = jnp.dot(wkT, pm_w2T_ref[...], preferred_element_type=f32)
    wpq_ref[...] = jnp.dot(pm_w2_ref[...], wq_ref[...], preferred_element_type=f32)
    pm_b2 = pm_b2_ref[...]
    cq_ref[...] = jnp.dot(pm_b2, wq_ref[...], preferred_element_type=f32) + bq_ref[...]
    ckb_ref[...] = jnp.dot(pm_b2, wkT_ref[...].T, preferred_element_type=f32) + bk_ref[...]
    co_ref[...] = jnp.dot(bv_ref[...], wo_ref[...], preferred_element_type=f32) + bo_ref[...]
    wv = wv_ref[...]
    wo = wo_ref[...]
    for h in range(NHEAD):
        hs = slice(h * DH, (h + 1) * DH)
        wvo_ref[h * D:(h + 1) * D, :] = jnp.dot(
            wv[:, hs], wo[hs, :], preferred_element_type=f32)


def _run_wprep(wkT, pm_w2T, pm_w2, wq, wv, wo, pm_b2, bq, bk, bv, bo):
    return pl.pallas_call(
        _wprep_body,
        out_shape=[
            jax.ShapeDtypeStruct((NHEAD * D, D), jnp.float32),
            jax.ShapeDtypeStruct((D, POS_HID), jnp.float32),
            jax.ShapeDtypeStruct((POS_HID, D), jnp.float32),
            jax.ShapeDtypeStruct((1, D), jnp.float32),
            jax.ShapeDtypeStruct((1, D), jnp.float32),
            jax.ShapeDtypeStruct((1, D), jnp.float32),
        ],
    )(wkT, pm_w2T, pm_w2, wq, wv, wo, pm_b2, bq, bk, bv, bo)


def _ln(x, g, b):
    m = jnp.mean(x, axis=-1, keepdims=True)
    xc = x - m
    v = jnp.mean(xc * xc, axis=-1, keepdims=True)
    return xc * lax.rsqrt(v + 1e-5) * g + b


def _attn_body(feats_ref, ownx_ref, gat_ref, d2_ref,
               wq_ref, wkT3_ref, wpkT3_ref, wpq_ref, cq_ref, ckb3_ref,
               wvo3_ref, co_ref, pm_w1_ref, pm_b1_ref, pm_g_ref, pm_bt_ref,
               l1w_ref, l1b_ref, l2w_ref, l2b_ref,
               n1g_ref, n1b_ref, n2g_ref, n2b_ref, out_ref,
               qh3_s, hmlp_s):
    f32 = jnp.float32
    R = ATT_R
    feats = feats_ref[...]
    gx = gat_ref[:, D:].reshape(R, K, XPAD)

    hpre = None
    for c in range(3):
        relc = gx[:, :, c:c + 1] - ownx_ref[:, c:c + 1].reshape(R, 1, 1)
        w1c = pm_w1_ref[c:c + 1, :].reshape(1, 1, POS_HID)
        term = (relc * (1.0 / 10.0)) * w1c
        hpre = term if hpre is None else hpre + term
    hpre = hpre + pm_b1_ref[...].reshape(1, 1, POS_HID)
    hm = jnp.mean(hpre, axis=-1, keepdims=True)
    hc = hpre - hm
    hv = jnp.mean(hc * hc, axis=-1, keepdims=True)
    hmlp = hc * lax.rsqrt(hv + 1e-5) * pm_g_ref[...].reshape(1, 1, POS_HID) \
        + pm_bt_ref[...].reshape(1, 1, POS_HID)
    hmlp = jnp.maximum(hmlp, 0.0)
    hmlp_s[...] = hmlp

    h0 = hmlp[:, 0, :]
    qh = (jnp.dot(feats, wq_ref[...], preferred_element_type=f32)
          + jnp.dot(h0, wpq_ref[...], preferred_element_type=f32)
          + cq_ref[...])
    for h in range(NHEAD):
        qh3_s[h] = qh[:, h * DH:(h + 1) * DH]

    mask = jnp.sqrt(d2_ref[...]) > 0.5
    scale = 1.0 / math.sqrt(float(DH))

    def head(h, o_acc):
        qh_h = qh3_s[h]
        u_h = jnp.dot(qh_h, wkT3_ref[h], preferred_element_type=f32)
        w_h = jnp.dot(qh_h, wpkT3_ref[h], preferred_element_type=f32)
        sb_h = jnp.sum(qh_h * ckb3_ref[h], axis=-1, keepdims=True)
        gf = gat_ref[:, :D].reshape(R, K, D)
        s_feat = jnp.sum(u_h[:, None, :] * gf, axis=-1)
        s_pos = jnp.sum(w_h[:, None, :] * hmlp_s[...], axis=-1)
        s = (s_feat + s_pos + sb_h) * scale
        s = jnp.where(mask, _NEG, s)
        smax = jnp.max(s, axis=-1, keepdims=True)
        e = jnp.exp(s - smax)
        attn = e / jnp.sum(e, axis=-1, keepdims=True)
        ctx_h = jnp.sum(attn[:, :, None] * gf, axis=1)
        return o_acc + jnp.dot(ctx_h, wvo3_ref[h], preferred_element_type=f32)

    o = lax.fori_loop(0, NHEAD, head, jnp.zeros((R, D), f32)) + co_ref[...]
    src = _ln(feats + o, n1g_ref[...], n1b_ref[...])
    ffp = jnp.dot(src, l1w_ref[...], preferred_element_type=f32) + l1b_ref[...]
    ff = ffp * 0.5 * (1.0 + lax.erf(ffp * (1.0 / math.sqrt(2.0))))
    ff = jnp.dot(ff, l2w_ref[...], preferred_element_type=f32) + l2b_ref[...]
    out_ref[...] = _ln(src + ff, n2g_ref[...], n2b_ref[...])


def _run_attn(feats, xyz_pad, gathered, d2,
              wq, wkT3, wpkT3, wpq, cq, ckb3, wvo3, co,
              pm_w1, pm_b1, pm_g, pm_bt, l1w, l1b, l2w, l2b,
              n1g, n1b, n2g, n2b):
    grid = N // ATT_R

    def full2(shape):
        return pl.BlockSpec(shape, lambda i: (0, 0))

    def full3(shape):
        return pl.BlockSpec(shape, lambda i: (0, 0, 0))

    return pl.pallas_call(
        _attn_body,
        grid=(grid,),
        in_specs=[
            pl.BlockSpec((ATT_R, D), lambda i: (i, 0)),
            pl.BlockSpec((ATT_R, 8), lambda i: (i, 0)),
            pl.BlockSpec((ATT_R * K, TW), lambda i: (i, 0)),
            pl.BlockSpec((ATT_R, K), lambda i: (i, 0)),
            full2((D, D)),
            full3((NHEAD, DH, D)),
            full3((NHEAD, DH, DH)),
            full2((POS_HID, D)),
            full2((1, D)),
            full3((NHEAD, 1, DH)),
            full3((NHEAD, D, D)),
            full2((1, D)),
            full2((3, POS_HID)),
            full2((1, POS_HID)),
            full2((1, POS_HID)),
            full2((1, POS_HID)),
            full2((D, D_FF)),
            full2((1, D_FF)),
            full2((D_FF, D)),
            full2((1, D)),
            full2((1, D)),
            full2((1, D)),
            full2((1, D)),
            full2((1, D)),
        ],
        out_specs=pl.BlockSpec((ATT_R, D), lambda i: (i, 0)),
        out_shape=jax.ShapeDtypeStruct((N, D), jnp.float32),
        scratch_shapes=[
            pltpu.VMEM((NHEAD, ATT_R, DH), jnp.float32),
            pltpu.VMEM((ATT_R, K, POS_HID), jnp.float32),
        ],
    )(feats, xyz_pad, gathered, d2,
      wq, wkT3, wpkT3, wpq, cq, ckb3, wvo3, co,
      pm_w1, pm_b1, pm_g, pm_bt, l1w, l1b, l2w, l2b,
      n1g, n1b, n2g, n2b)


def kernel(pts_feats, pts_xyz, pts_inds, pm_w1, pm_b1, pm_g, pm_bt, pm_w2,
           pm_b2, wq, bq, wk, bk, wv, bv, wo, bo, l1w, l1b, l2w, l2b,
           n1g, n1b, n2g, n2b):
    del pts_inds

    def row(v):
        return v.reshape(1, -1)

    xyz_pad8 = jnp.pad(pts_xyz, ((0, 0), (0, 8 - 3)))
    xyz_t = jnp.pad(pts_xyz.T, ((0, 8 - 3), (0, 0)))

    inds, d2 = _run_knn(xyz_pad8, xyz_t)

    table = jnp.concatenate(
        [pts_feats, jnp.pad(pts_xyz, ((0, 0), (0, XPAD - 3)))], axis=1)
    gathered = _run_sc_gather(table, inds.reshape(-1))

    wvo, wpkT, wpq, cq, ckb, co = _run_wprep(
        wk.T, pm_w2.T, pm_w2, wq, wv, wo,
        row(pm_b2), row(bq), row(bk), row(bv), row(bo))

    wkT3 = wk.T.reshape(NHEAD, DH, D)
    wpkT3 = wpkT.reshape(NHEAD, DH, DH)
    ckb3 = ckb.reshape(NHEAD, 1, DH)
    wvo3 = wvo.reshape(NHEAD, D, D)

    return _run_attn(
        pts_feats, xyz_pad8, gathered, d2,
        wq, wkT3, wpkT3, wpq, cq, ckb3, wvo3, co,
        pm_w1, row(pm_b1), row(pm_g), row(pm_bt),
        l1w, row(l1b), l2w, row(l2b),
        row(n1g), row(n1b), row(n2g), row(n2b))
